# Initial kernel scaffold; baseline (speedup 1.0000x reference)
import jax
import jax.numpy as jnp
from jax.experimental import pallas as pl


def kernel(x_G, x_R, Wk, bk, Wq, bq, Wv, bv, Ws, bs, Wg, bg, ln_g, ln_b, layer_weights, W_las, ei_g_to_r, ei_r_to_r):
    raise NotImplementedError("write your pallas kernel here")


# speedup vs baseline: 8.4097x; 8.4097x over previous
import functools

import jax
import jax.numpy as jnp
from jax import lax
from jax.experimental import pallas as pl
from jax.experimental.pallas import tpu as pltpu
from jax.experimental.pallas import tpu_sc as plsc

B = 2
NG = 10000
NR = 10000
GE = 128
RE = 256
E = 160000
HALF = 128
BN = B * NR

NT = 16
C = 16
SEG = 25
SR = 25
SPT = 624
TAIL = NR - NT * SPT
Q = 2
QW = RE // Q
NHALF = NR // 2
ACCR = NHALF + 8
APT = NHALF // NT
ATAIL = NHALF - NT * APT

R_BLK = 1000
GRID = BN // R_BLK

@functools.cache
def _mesh():
    return plsc.VectorSubcoreMesh(core_axis_name="c", subcore_axis_name="s")


def _deg_body(cidx, out, idx_v, ones_v, dbuf_v, acc_s):
    c = lax.axis_index("c")
    s = lax.axis_index("s")
    nbase = c * NHALF

    def zrow(i, _):
        dbuf_v[i, :] = jnp.zeros((16,), jnp.float32)
        return 0
    lax.fori_loop(0, APT, zrow, 0)

    def orow(i, _):
        ones_v[i, :] = jnp.ones((16,), jnp.float32)
        return 0
    lax.fori_loop(0, C, orow, 0)

    pltpu.sync_copy(dbuf_v, acc_s.at[pl.ds(s * APT, APT)])

    @pl.when(s == 0)
    def _():
        pltpu.sync_copy(dbuf_v.at[pl.ds(0, ATAIL)],
                        acc_s.at[pl.ds(NT * APT, ATAIL)])

    plsc.subcore_barrier()

    def seg_loop(g, _):
        pltpu.sync_copy(cidx.at[s, g], idx_v)

        def fix(j, _):
            for k in range(C // 16):
                sl = pl.ds(k * 16, 16)
                t = idx_v[j, sl] - nbase
                ok = (t >= 0) & (t < NHALF)
                idx_v[j, sl] = jnp.where(ok, t, NHALF)
            return 0
        lax.fori_loop(0, SR, fix, 0)

        def chunk(j, _):
            pltpu.sync_copy(ones_v, acc_s.at[idx_v.at[j]], add=True)
            return 0
        lax.fori_loop(0, SR, chunk, 0)
        return 0
    lax.fori_loop(0, SEG, seg_loop, 0)
    plsc.subcore_barrier()

    pltpu.sync_copy(acc_s.at[pl.ds(s * APT, APT)], dbuf_v)
    pltpu.sync_copy(dbuf_v, out.at[pl.ds(nbase + s * APT, APT)])

    @pl.when(s == 0)
    def _():
        pltpu.sync_copy(acc_s.at[pl.ds(NT * APT, ATAIL)],
                        dbuf_v.at[pl.ds(0, ATAIL)])
        pltpu.sync_copy(dbuf_v.at[pl.ds(0, ATAIL)],
                        out.at[pl.ds(nbase + NT * APT, ATAIL)])


@functools.cache
def _deg_call():
    return pl.kernel(
        _deg_body,
        out_type=jax.ShapeDtypeStruct((NR, 16), jnp.float32),
        mesh=_mesh(),
        scratch_types=[
            pltpu.VMEM((SR, C), jnp.int32),
            pltpu.VMEM((C, 16), jnp.float32),
            pltpu.VMEM((APT, 16), jnp.float32),
            pltpu.VMEM_SHARED((ACCR, 16), jnp.float32),
        ],
    )


def _edge_body(ktab, qvtab, hptab, kidx, qidx, hidx, didx, cidx,
               aggR, aggG, idxg, idxg2, idxs,
               g0, g1, qv0, qv1, dbuf, acc_s, sk0, sk1, sq0, sq1):
    c = lax.axis_index("c")
    s = lax.axis_index("s")
    grows = (g0, g1)
    qvrows = (qv0, qv1)
    semks = (sk0, sk1)
    semqs = (sq0, sq1)

    def zdbuf(i, _):
        for kk in range(QW // 16):
            dbuf[i, pl.ds(kk * 16, 16)] = jnp.zeros((16,), jnp.float32)
        return 0
    lax.fori_loop(0, 24, zdbuf, 0)

    def zero_acc():
        def zc(m, _):
            pltpu.sync_copy(dbuf, acc_s.at[pl.ds(s * SPT + m * 24, 24)])
            return 0
        lax.fori_loop(0, SPT // 24, zc, 0)

        @pl.when(s == 0)
        def _():
            pltpu.sync_copy(dbuf.at[pl.ds(0, TAIL)],
                            acc_s.at[pl.ds(NT * SPT, TAIL)])

    def drain(outref, h):
        base_out = c * NR + s * SPT
        for m in range(SPT // 24):
            pltpu.sync_copy(acc_s.at[pl.ds(s * SPT + m * 24, 24)], dbuf)
            pltpu.sync_copy(dbuf, outref.at[h, pl.ds(base_out + m * 24, 24)])

        @pl.when(s == 0)
        def _():
            pltpu.sync_copy(acc_s.at[pl.ds(NT * SPT, TAIL)],
                            dbuf.at[pl.ds(0, TAIL)])
            pltpu.sync_copy(dbuf.at[pl.ds(0, TAIL)],
                            outref.at[h, pl.ds(c * NR + NT * SPT, TAIL)])
        lax.fori_loop(0, 24, zdbuf, 0)

    def gathers_start(tab, idxref, u, p, sems, bufs):
        pltpu.async_copy(tab.at[idxref.at[u]], bufs[p], sems[p])

    def gather_wait(tab, sems, bufs, p):
        pltpu.make_async_copy(tab.at[idxg.at[0]], bufs[p], sems[p]).wait()

    def compute_msg(p):
        def row(r, _):
            for kk in range(QW // 16):
                sl = pl.ds(kk * 16, 16)
                x = grows[p][r, sl] + qvrows[p][r, sl]
                grows[p][r, sl] = qvrows[p][r, pl.ds(QW + kk * 16, 16)] / (
                    1.0 + jnp.exp(-x))
            return 0
        lax.fori_loop(0, C, row, 0)

    for h in range(Q):
        zero_acc()
        plsc.subcore_barrier()

        def rg_seg(g, _):
            pltpu.sync_copy(didx.at[s, g], idxs)
            pltpu.sync_copy(kidx.at[h, c, s, g], idxg)
            pltpu.sync_copy(qidx.at[h, c, s, g], idxg2)
            gathers_start(ktab, idxg, 0, 0, semks, grows)
            gathers_start(qvtab, idxg2, 0, 0, semqs, qvrows)

            def rg_pair(t, _):
                for i in range(2):
                    u = 2 * t + i
                    gather_wait(ktab, semks, grows, i)
                    gather_wait(qvtab, semqs, qvrows, i)
                    gathers_start(ktab, idxg, u + 1, 1 - i, semks, grows)
                    gathers_start(qvtab, idxg2, u + 1, 1 - i, semqs, qvrows)
                    compute_msg(i)
                    pltpu.sync_copy(grows[i], acc_s.at[idxs.at[u]], add=True)
                return 0
            lax.fori_loop(0, (SR - 1) // 2, rg_pair, 0)
            gather_wait(ktab, semks, grows, 0)
            gather_wait(qvtab, semqs, qvrows, 0)
            compute_msg(0)
            pltpu.sync_copy(grows[0], acc_s.at[idxs.at[SR - 1]], add=True)
            return 0
        lax.fori_loop(0, SEG, rg_seg, 0)
        plsc.subcore_barrier()
        drain(aggR, h)

        zero_acc()
        plsc.subcore_barrier()

        def gc_seg(g, _):
            pltpu.sync_copy(cidx.at[s, g], idxs)
            pltpu.sync_copy(hidx.at[h, c, s, g], idxg)
            gathers_start(hptab, idxg, 0, 0, semks, grows)

            def gc_pair(t, _):
                for i in range(2):
                    u = 2 * t + i
                    gather_wait(hptab, semks, grows, i)
                    gathers_start(hptab, idxg, u + 1, 1 - i, semks, grows)
                    pltpu.sync_copy(grows[i], acc_s.at[idxs.at[u]], add=True)
                return 0
            lax.fori_loop(0, (SR - 1) // 2, gc_pair, 0)
            gather_wait(hptab, semks, grows, 0)
            pltpu.sync_copy(grows[0], acc_s.at[idxs.at[SR - 1]], add=True)
            return 0
        lax.fori_loop(0, SEG, gc_seg, 0)
        plsc.subcore_barrier()
        drain(aggG, h)


@functools.cache
def _edge_call():
    return pl.kernel(
        _edge_body,
        out_type=[
            jax.ShapeDtypeStruct((Q, BN, QW), jnp.float32),
            jax.ShapeDtypeStruct((Q, BN, QW), jnp.float32),
        ],
        mesh=_mesh(),
        scratch_types=[
            pltpu.VMEM((SR, C), jnp.int32),
            pltpu.VMEM((SR, C), jnp.int32),
            pltpu.VMEM((SR, C), jnp.int32),
            pltpu.VMEM((C, QW), jnp.float32),
            pltpu.VMEM((C, QW), jnp.float32),
            pltpu.VMEM((C, 2 * QW), jnp.float32),
            pltpu.VMEM((C, 2 * QW), jnp.float32),
            pltpu.VMEM((24, QW), jnp.float32),
            pltpu.VMEM_SHARED((NR, QW), jnp.float32),
        ] + [pltpu.SemaphoreType.DMA] * 4,
    )


def _p0_body(xr, xg, degw, Wk0, bk0, Ws0, bs0, Wg0, Wq0, bq0, Wv0, bv0,
             Wq1, bq1, Wv1, bv1, k0t, s0o, hp0t, qv0t, qv1t):
    x = xr[...]
    g = xg[...]
    dis = lax.rsqrt(degw[:, 0] + 1.0)
    k0 = jnp.dot(x, Wk0[...]) + bk0[0]
    s0 = jnp.dot(x, Ws0[...]) + bs0[0]
    hp = jnp.dot(x, Wg0[...]) * dis[:, None]
    q0 = jnp.dot(g, Wq0[...]) + bq0[0]
    v0 = jnp.dot(g, Wv0[...]) + bv0[0]
    q1 = jnp.dot(g, Wq1[...]) + bq1[0]
    v1 = jnp.dot(g, Wv1[...]) + bv1[0]
    s0o[...] = s0
    for qq in range(Q):
        sl = slice(qq * QW, (qq + 1) * QW)
        k0t[qq] = k0[:, sl]
        hp0t[qq] = hp[:, sl]
        qv0t[qq, :, :QW] = q0[:, sl]
        qv0t[qq, :, QW:] = v0[:, sl]
        qv1t[qq, :, :QW] = q1[:, sl]
        qv1t[qq, :, QW:] = v1[:, sl]


def _softmax3(lwr):
    l0, l1, l2 = lwr[0, 0], lwr[0, 1], lwr[0, 2]
    mx = jnp.maximum(jnp.maximum(l0, l1), l2)
    e0, e1, e2 = jnp.exp(l0 - mx), jnp.exp(l1 - mx), jnp.exp(l2 - mx)
    den = e0 + e1 + e2
    return e0 / den, e1 / den, e2 / den


def _combine(s_ref, aR, aG, hpt, degw, bg, lng, lnb):
    dis = lax.rsqrt(degw[:, 0] + 1.0)
    aggR = jnp.concatenate([aR[qq] for qq in range(Q)], axis=1)
    aggG = jnp.concatenate([aG[qq] for qq in range(Q)], axis=1)
    hp = jnp.concatenate([hpt[qq] for qq in range(Q)], axis=1)
    out = s_ref[...] + aggR + (aggG + hp) * dis[:, None] + bg[0]
    y = 0.5 * out * (1.0 + lax.erf(out * 0.7071067811865476))
    m = jnp.mean(y, axis=1, keepdims=True)
    v = jnp.mean((y - m) ** 2, axis=1, keepdims=True)
    return (y - m) / jnp.sqrt(v + 1e-5) * lng[0] + lnb[0], dis


def _u0_body(s0, aR, aG, hpt, degw, xr0, lwr, bg0, lng0, lnb0,
             Wk1, bk1, Ws1, bs1, Wg1, k1t, s1o, hp1t, rr1o):
    xr1, dis = _combine(s0, aR, aG, hpt, degw, bg0, lng0, lnb0)
    w0, w1, _ = _softmax3(lwr)
    rr1o[...] = w0 * xr0[...] + w1 * xr1
    k1 = jnp.dot(xr1, Wk1[...]) + bk1[0]
    s1o[...] = jnp.dot(xr1, Ws1[...]) + bs1[0]
    hp1 = jnp.dot(xr1, Wg1[...]) * dis[:, None]
    for qq in range(Q):
        sl = slice(qq * QW, (qq + 1) * QW)
        k1t[qq] = k1[:, sl]
        hp1t[qq] = hp1[:, sl]


def _u1_body(s1, aR, aG, hpt, degw, rr1, lwr, bg1, lng1, lnb1, Wlas, flxo):
    xr2, _ = _combine(s1, aR, aG, hpt, degw, bg1, lng1, lnb1)
    _, _, w2 = _softmax3(lwr)
    rr = rr1[...] + w2 * xr2
    a = rr[:, :HALF]
    cc = rr[:, HALF:]
    flx = (jnp.sum(jnp.dot(a, Wlas[...]) * cc, axis=1)
           - jnp.sum(jnp.dot(cc, Wlas[...]) * a, axis=1))
    flxo[...] = jnp.broadcast_to(flx[:, None], (R_BLK, HALF))


def _row_spec(w):
    return pl.BlockSpec((R_BLK, w), lambda i: (i, 0))


def _full_spec(shape):
    nd = len(shape)
    return pl.BlockSpec(shape, lambda i: (0,) * nd)


def _q_spec(w):
    return pl.BlockSpec((Q, R_BLK, w), lambda i: (0, i, 0))


_DEG_SPEC = pl.BlockSpec((R_BLK, 16), lambda i: (i % (NR // R_BLK), 0))


def _p0_call(xr, xg, degw, Wk0, bk0, Ws0, bs0, Wg0, Wq0, bq0, Wv0, bv0,
             Wq1, bq1, Wv1, bv1):
    return pl.pallas_call(
        _p0_body,
        grid=(GRID,),
        in_specs=[
            _row_spec(RE), _row_spec(GE), _DEG_SPEC,
            _full_spec((RE, RE)), _full_spec((1, RE)),
            _full_spec((RE, RE)), _full_spec((1, RE)),
            _full_spec((RE, RE)),
            _full_spec((GE, RE)), _full_spec((1, RE)),
            _full_spec((GE, RE)), _full_spec((1, RE)),
            _full_spec((GE, RE)), _full_spec((1, RE)),
            _full_spec((GE, RE)), _full_spec((1, RE)),
        ],
        out_specs=[
            _q_spec(QW), _row_spec(RE), _q_spec(QW),
            _q_spec(2 * QW), _q_spec(2 * QW),
        ],
        out_shape=[
            jax.ShapeDtypeStruct((Q, BN, QW), jnp.float32),
            jax.ShapeDtypeStruct((BN, RE), jnp.float32),
            jax.ShapeDtypeStruct((Q, BN, QW), jnp.float32),
            jax.ShapeDtypeStruct((Q, BN, 2 * QW), jnp.float32),
            jax.ShapeDtypeStruct((Q, BN, 2 * QW), jnp.float32),
        ],
    )(xr, xg, degw, Wk0, bk0, Ws0, bs0, Wg0, Wq0, bq0, Wv0, bv0,
      Wq1, bq1, Wv1, bv1)


def _u0_call(s0, aggR, aggG, hp0t, degw, xr0, lwr, bg0, lng0, lnb0,
             Wk1, bk1, Ws1, bs1, Wg1):
    return pl.pallas_call(
        _u0_body,
        grid=(GRID,),
        in_specs=[
            _row_spec(RE), _q_spec(QW), _q_spec(QW),
            _q_spec(QW), _DEG_SPEC, _row_spec(RE),
            _full_spec((1, 3)), _full_spec((1, RE)),
            _full_spec((1, RE)), _full_spec((1, RE)),
            _full_spec((RE, RE)), _full_spec((1, RE)),
            _full_spec((RE, RE)), _full_spec((1, RE)),
            _full_spec((RE, RE)),
        ],
        out_specs=[
            _q_spec(QW), _row_spec(RE), _q_spec(QW), _row_spec(RE),
        ],
        out_shape=[
            jax.ShapeDtypeStruct((Q, BN, QW), jnp.float32),
            jax.ShapeDtypeStruct((BN, RE), jnp.float32),
            jax.ShapeDtypeStruct((Q, BN, QW), jnp.float32),
            jax.ShapeDtypeStruct((BN, RE), jnp.float32),
        ],
    )(s0, aggR, aggG, hp0t, degw, xr0, lwr, bg0, lng0, lnb0,
      Wk1, bk1, Ws1, bs1, Wg1)


def _u1_call(s1, aggR, aggG, hp1t, degw, rr1, lwr, bg1, lng1, lnb1, Wlas):
    return pl.pallas_call(
        _u1_body,
        grid=(GRID,),
        in_specs=[
            _row_spec(RE), _q_spec(QW), _q_spec(QW),
            _q_spec(QW), _DEG_SPEC, _row_spec(RE),
            _full_spec((1, 3)), _full_spec((1, RE)),
            _full_spec((1, RE)), _full_spec((1, RE)),
            _full_spec((HALF, HALF)),
        ],
        out_specs=[_row_spec(HALF)],
        out_shape=[jax.ShapeDtypeStruct((BN, HALF), jnp.float32)],
    )(s1, aggR, aggG, hp1t, degw, rr1, lwr, bg1, lng1, lnb1, Wlas)


def kernel(x_G, x_R, Wk, bk, Wq, bq, Wv, bv, Ws, bs, Wg, bg, ln_g, ln_b,
           layer_weights, W_las, ei_g_to_r, ei_r_to_r):
    s_gr = ei_g_to_r[0].astype(jnp.int32)
    d_gr = ei_g_to_r[1].astype(jnp.int32)
    r_rr = ei_r_to_r[0].astype(jnp.int32)
    c_rr = ei_r_to_r[1].astype(jnp.int32)

    def gather_idx(idx, n):
        base = idx.reshape(NT, SEG, SR, C)
        return jnp.stack([
            jnp.stack([base + (hh * B + b) * n for b in range(B)])
            for hh in range(Q)])

    kidx = gather_idx(d_gr, NR)
    qidx = gather_idx(s_gr, NG)
    hidx = gather_idx(r_rr, NR)
    didx = d_gr.reshape(NT, SEG, SR, C)
    cidx = c_rr.reshape(NT, SEG, SR, C)

    xr2 = x_R.reshape(BN, RE)
    xg2 = x_G.reshape(B * NG, GE)
    lwr = layer_weights.reshape(1, 3)
    b2 = lambda t: t.reshape(1, RE)

    degw = _deg_call()(cidx)

    k0t, s0, hp0t, qv0t, qv1t = _p0_call(
        xr2, xg2, degw, Wk[0], b2(bk[0]), Ws[0], b2(bs[0]), Wg[0],
        Wq[0], b2(bq[0]), Wv[0], b2(bv[0]), Wq[1], b2(bq[1]),
        Wv[1], b2(bv[1]))

    aggR0, aggG0 = _edge_call()(
        k0t.reshape(Q * BN, QW), qv0t.reshape(Q * BN, 2 * QW),
        hp0t.reshape(Q * BN, QW), kidx, qidx, hidx, didx, cidx)

    k1t, s1, hp1t, rr1 = _u0_call(
        s0, aggR0, aggG0, hp0t, degw, xr2, lwr, b2(bg[0]), b2(ln_g[0]),
        b2(ln_b[0]), Wk[1], b2(bk[1]), Ws[1], b2(bs[1]), Wg[1])

    aggR1, aggG1 = _edge_call()(
        k1t.reshape(Q * BN, QW), qv1t.reshape(Q * BN, 2 * QW),
        hp1t.reshape(Q * BN, QW), kidx, qidx, hidx, didx, cidx)

    (flxw,) = _u1_call(
        s1, aggR1, aggG1, hp1t, degw, rr1, lwr, b2(bg[1]), b2(ln_g[1]),
        b2(ln_b[1]), W_las)

    return flxw[:, 0].reshape(B, NR)
